# baseline (speedup 1.0000x reference)
import functools

import jax
import jax.numpy as jnp
from jax import lax
from jax.experimental import pallas as pl
from jax.experimental.pallas import tpu as pltpu
from jax.experimental.pallas import tpu_sc as plsc

N_NODES = 100000
NUM_TYPES = 128
L = 16
NC, NS = 2, 16
NW = NC * NS
C = 128
FULL_CHUNKS = N_NODES // C
TAIL = N_NODES - FULL_CHUNKS * C
MAX_CHUNKS_PER_WORKER = (FULL_CHUNKS + 1 + NW - 1) // NW

_mesh = plsc.VectorSubcoreMesh(core_axis_name="c", subcore_axis_name="s")


@functools.partial(
    pl.kernel,
    mesh=_mesh,
    compiler_params=pltpu.CompilerParams(needs_layout_passes=False),
    out_type=jax.ShapeDtypeStruct((N_NODES, NUM_TYPES), jnp.float32),
    scratch_types=[
        pltpu.VMEM((C,), jnp.int32),
        pltpu.VMEM((C, NUM_TYPES), jnp.float32),
    ],
)
def _onehot_sc(idx_hbm, zeros_hbm, out_hbm, idx_v, buf):
    wid = lax.axis_index("s") * NC + lax.axis_index("c")

    lane = lax.iota(jnp.int32, L)
    ones = jnp.full((L,), 1.0, jnp.float32)
    zeros = jnp.full((L,), 0.0, jnp.float32)

    pltpu.sync_copy(zeros_hbm, buf)

    def scatter(rows, value):
        for g in range(rows // L):
            iv = idx_v[pl.ds(g * L, L)]
            plsc.store_scatter(buf, [g * L + lane, iv], value)

    def do_chunk(c_id, rows):
        base = c_id * C
        pltpu.sync_copy(idx_hbm.at[pl.ds(base, rows)], idx_v.at[pl.ds(0, rows)])
        scatter(rows, ones)
        pltpu.sync_copy(buf.at[pl.ds(0, rows)],
                        out_hbm.at[pl.ds(base, rows)])
        scatter(rows, zeros)

    def body(k, carry):
        c_id = wid + NW * k

        @pl.when(c_id < FULL_CHUNKS)
        def _():
            do_chunk(c_id, C)

        @pl.when(c_id == FULL_CHUNKS)
        def _():
            do_chunk(c_id, TAIL)

        return carry

    lax.fori_loop(0, MAX_CHUNKS_PER_WORKER, body, 0)


def kernel(atom_type, pos):
    idx = atom_type.reshape(-1).astype(jnp.int32)
    zeros_init = jnp.zeros((C, NUM_TYPES), jnp.float32)
    out = _onehot_sc(idx, zeros_init)
    return (out, out)


# speedup vs baseline: 1.0378x; 1.0378x over previous
import functools

import jax
import jax.numpy as jnp
from jax import lax
from jax.experimental import pallas as pl
from jax.experimental.pallas import tpu as pltpu
from jax.experimental.pallas import tpu_sc as plsc

N_NODES = 100000
NUM_TYPES = 128
L = 16
NC, NS = 2, 16
NW = NC * NS
C = 400
NCHUNKS = N_NODES // C
MAXK = (NCHUNKS + NW - 1) // NW

_mesh = plsc.VectorSubcoreMesh(core_axis_name="c", subcore_axis_name="s")


@functools.partial(
    pl.kernel,
    mesh=_mesh,
    compiler_params=pltpu.CompilerParams(needs_layout_passes=False),
    out_type=jax.ShapeDtypeStruct((N_NODES, NUM_TYPES), jnp.float32),
    scratch_types=[
        pltpu.VMEM((C,), jnp.int32),
        pltpu.VMEM((C,), jnp.int32),
        pltpu.VMEM((C, NUM_TYPES), jnp.float32),
        pltpu.VMEM((C, NUM_TYPES), jnp.float32),
        pltpu.SemaphoreType.DMA,
        pltpu.SemaphoreType.DMA,
    ],
)
def _onehot_sc(idx_hbm, zeros_hbm, out_hbm, idx0, idx1, buf0, buf1, sem0, sem1):
    wid = lax.axis_index("s") * NC + lax.axis_index("c")
    idxs, bufs, sems = (idx0, idx1), (buf0, buf1), (sem0, sem1)

    lane = lax.iota(jnp.int32, L)
    ones = jnp.full((L,), 1.0, jnp.float32)
    zeros = jnp.full((L,), 0.0, jnp.float32)

    def scatter(buf, idx_v, value):
        for g in range(C // L):
            iv = idx_v[pl.ds(g * L, L)]
            plsc.store_scatter(buf, [g * L + lane, iv], value)

    pltpu.async_copy(zeros_hbm, buf0, sem0)
    pltpu.async_copy(zeros_hbm, buf1, sem1)

    for k in range(MAXK):
        b = k % 2
        c = wid + NW * k

        @pl.when(c < NCHUNKS)
        def _(k=k, b=b, c=c):
            if k < 2:
                pltpu.make_async_copy(zeros_hbm, bufs[b], sems[b]).wait()
            else:
                pltpu.make_async_copy(
                    bufs[b], out_hbm.at[pl.ds((c - 2 * NW) * C, C)], sems[b]
                ).wait()
                scatter(bufs[b], idxs[b], zeros)
            pltpu.sync_copy(idx_hbm.at[pl.ds(c * C, C)], idxs[b])
            scatter(bufs[b], idxs[b], ones)
            pltpu.async_copy(bufs[b], out_hbm.at[pl.ds(c * C, C)], sems[b])

    pltpu.make_async_copy(buf0, out_hbm.at[pl.ds(0, C)], sem0).wait()
    pltpu.make_async_copy(buf1, out_hbm.at[pl.ds(0, C)], sem1).wait()


def kernel(atom_type, pos):
    idx = atom_type.reshape(-1).astype(jnp.int32)
    zeros_init = jnp.zeros((C, NUM_TYPES), jnp.float32)
    out = _onehot_sc(idx, zeros_init)
    return (out, out)
